# Initial kernel scaffold; baseline (speedup 1.0000x reference)
import jax
import jax.numpy as jnp
from jax.experimental import pallas as pl


def kernel(x, edge_index, edge_attr, W_in, b_in, W_edge, W_h, b_h, W_o, b_o):
    raise NotImplementedError("write your pallas kernel here")


# speedup vs baseline: 2.4568x; 2.4568x over previous
import functools

import jax
import jax.numpy as jnp
from jax import lax
from jax.experimental import pallas as pl
from jax.experimental.pallas import tpu as pltpu
from jax.experimental.pallas import tpu_sc as plsc

N = 10000
E = 320000
D_ATOM = 128
D_BOND = 16
HIDDEN = 128

NC = 2
NS = 16
NW = NC * NS

CHUNK = 128
CHUNKS_PER_TILE = 80
EDGES_PER_TILE = CHUNK * CHUNKS_PER_TILE
E_PAD = EDGES_PER_TILE * NW
N_ACC = N + 16
ZROWS_ACC = N_ACC // NS
OROWS = N // NS

_mesh = plsc.VectorSubcoreMesh(
    core_axis_name="c", subcore_axis_name="s", num_cores=NC, num_subcores=NS)


def _zero_vmem(ref, nrows, ncol):
  z = jnp.zeros((16,), jnp.float32)
  @pl.loop(0, nrows)
  def _(i):
    for j in range(ncol // 16):
      ref[i, pl.ds(j * 16, 16)] = z


def _sc_gather_segsum(h_hbm, src_hbm, dst_hbm, out_hbm,
                      sidx, didx, rows, acc, sem):
  cid = lax.axis_index("c")
  sid = lax.axis_index("s")
  wid = cid * NS + sid

  _zero_vmem(rows, CHUNK, HIDDEN)
  zbase = sid * ZROWS_ACC
  for k in range(ZROWS_ACC // CHUNK):
    pltpu.sync_copy(rows, acc.at[pl.ds(zbase + k * CHUNK, CHUNK)])
  rem = ZROWS_ACC % CHUNK
  if rem:
    pltpu.sync_copy(rows.at[pl.ds(0, rem)],
                    acc.at[pl.ds(zbase + (ZROWS_ACC // CHUNK) * CHUNK, rem)])

  rbase = wid * CHUNKS_PER_TILE
  pltpu.sync_copy(src_hbm.at[pl.ds(rbase, CHUNKS_PER_TILE)], sidx)
  pltpu.sync_copy(dst_hbm.at[pl.ds(rbase, CHUNKS_PER_TILE)], didx)

  plsc.subcore_barrier()

  @pl.loop(0, CHUNKS_PER_TILE)
  def _(j):
    pltpu.async_copy(h_hbm.at[sidx.at[j]], rows, sem).wait()
    pltpu.sync_copy(rows, acc.at[didx.at[j]], add=True)

  plsc.subcore_barrier()

  obase = sid * OROWS
  pltpu.sync_copy(acc.at[pl.ds(obase, OROWS)],
                  out_hbm.at[cid, pl.ds(obase, OROWS)])


_gather_segsum = pl.kernel(
    _sc_gather_segsum,
    out_type=jax.ShapeDtypeStruct((NC, N, HIDDEN), jnp.float32),
    mesh=_mesh,
    compiler_params=pltpu.CompilerParams(use_tc_tiling_on_sc=False),
    scratch_types=[
        pltpu.VMEM((CHUNKS_PER_TILE, CHUNK), jnp.int32),
        pltpu.VMEM((CHUNKS_PER_TILE, CHUNK), jnp.int32),
        pltpu.VMEM((CHUNK, HIDDEN), jnp.float32),
        pltpu.VMEM_SHARED((N_ACC, HIDDEN), jnp.float32),
        pltpu.SemaphoreType.DMA,
    ],
)


def _sc_edge_segsum(ea_hbm, dst_hbm, out_hbm, didx, rows, acc):
  cid = lax.axis_index("c")
  sid = lax.axis_index("s")
  wid = cid * NS + sid

  _zero_vmem(rows, CHUNK, D_BOND)
  zbase = sid * ZROWS_ACC
  for k in range(ZROWS_ACC // CHUNK):
    pltpu.sync_copy(rows, acc.at[pl.ds(zbase + k * CHUNK, CHUNK)])
  rem = ZROWS_ACC % CHUNK
  if rem:
    pltpu.sync_copy(rows.at[pl.ds(0, rem)],
                    acc.at[pl.ds(zbase + (ZROWS_ACC // CHUNK) * CHUNK, rem)])

  rbase = wid * CHUNKS_PER_TILE
  pltpu.sync_copy(dst_hbm.at[pl.ds(rbase, CHUNKS_PER_TILE)], didx)

  plsc.subcore_barrier()

  ebase = wid * EDGES_PER_TILE
  @pl.loop(0, CHUNKS_PER_TILE)
  def _(j):
    pltpu.sync_copy(ea_hbm.at[pl.ds(ebase + j * CHUNK, CHUNK)], rows)
    pltpu.sync_copy(rows, acc.at[didx.at[j]], add=True)

  plsc.subcore_barrier()

  obase = sid * OROWS
  pltpu.sync_copy(acc.at[pl.ds(obase, OROWS)],
                  out_hbm.at[cid, pl.ds(obase, OROWS)])


_edge_segsum = pl.kernel(
    _sc_edge_segsum,
    out_type=jax.ShapeDtypeStruct((NC, N, D_BOND), jnp.float32),
    mesh=_mesh,
    compiler_params=pltpu.CompilerParams(use_tc_tiling_on_sc=False),
    scratch_types=[
        pltpu.VMEM((CHUNKS_PER_TILE, CHUNK), jnp.int32),
        pltpu.VMEM((CHUNK, D_BOND), jnp.float32),
        pltpu.VMEM_SHARED((N_ACC, D_BOND), jnp.float32),
    ],
)


_ROWS_BLK = 1000
_GRID = N // _ROWS_BLK


def _tc_pre_body(x_ref, ea_ref, win_ref, bin_ref, wedge_ref, h0_ref, eagg_ref):
  h0 = jnp.maximum(
      jnp.dot(x_ref[...], win_ref[...], preferred_element_type=jnp.float32)
      + bin_ref[...], 0.0)
  h0_ref[...] = h0
  ea = ea_ref[0] + ea_ref[1]
  eagg_ref[...] = jnp.dot(ea, wedge_ref[...],
                          preferred_element_type=jnp.float32)


def _tc_pre(x, ea2, W_in, b_in, W_edge):
  return pl.pallas_call(
      _tc_pre_body,
      grid=(_GRID,),
      in_specs=[
          pl.BlockSpec((_ROWS_BLK, D_ATOM), lambda i: (i, 0)),
          pl.BlockSpec((NC, _ROWS_BLK, D_BOND), lambda i: (0, i, 0)),
          pl.BlockSpec((D_ATOM, HIDDEN), lambda i: (0, 0)),
          pl.BlockSpec((1, HIDDEN), lambda i: (0, 0)),
          pl.BlockSpec((D_BOND, HIDDEN), lambda i: (0, 0)),
      ],
      out_specs=[
          pl.BlockSpec((_ROWS_BLK, HIDDEN), lambda i: (i, 0)),
          pl.BlockSpec((_ROWS_BLK, HIDDEN), lambda i: (i, 0)),
      ],
      out_shape=[
          jax.ShapeDtypeStruct((N, HIDDEN), jnp.float32),
          jax.ShapeDtypeStruct((N, HIDDEN), jnp.float32),
      ],
  )(x, ea2, W_in, b_in, W_edge)


def _tc_step_body(acc_ref, eagg_ref, h0_ref, wh_ref, bh_ref, h_ref):
  agg = acc_ref[0] + acc_ref[1] + eagg_ref[...]
  h_ref[...] = jnp.maximum(
      jnp.dot(agg, wh_ref[...], preferred_element_type=jnp.float32)
      + bh_ref[...] + h0_ref[...], 0.0)


def _tc_step(acc, eagg, h0, W_h, b_h):
  return pl.pallas_call(
      _tc_step_body,
      grid=(_GRID,),
      in_specs=[
          pl.BlockSpec((NC, _ROWS_BLK, HIDDEN), lambda i: (0, i, 0)),
          pl.BlockSpec((_ROWS_BLK, HIDDEN), lambda i: (i, 0)),
          pl.BlockSpec((_ROWS_BLK, HIDDEN), lambda i: (i, 0)),
          pl.BlockSpec((HIDDEN, HIDDEN), lambda i: (0, 0)),
          pl.BlockSpec((1, HIDDEN), lambda i: (0, 0)),
      ],
      out_specs=pl.BlockSpec((_ROWS_BLK, HIDDEN), lambda i: (i, 0)),
      out_shape=jax.ShapeDtypeStruct((N, HIDDEN), jnp.float32),
  )(acc, eagg, h0, W_h, b_h)


def _tc_out_body(x_ref, h_ref, wo1_ref, wo2_ref, bo_ref, out_ref):
  out_ref[...] = jnp.maximum(
      jnp.dot(x_ref[...], wo1_ref[...], preferred_element_type=jnp.float32)
      + jnp.dot(h_ref[...], wo2_ref[...], preferred_element_type=jnp.float32)
      + bo_ref[...], 0.0)


def _tc_out(x, h, W_o1, W_o2, b_o):
  return pl.pallas_call(
      _tc_out_body,
      grid=(_GRID,),
      in_specs=[
          pl.BlockSpec((_ROWS_BLK, D_ATOM), lambda i: (i, 0)),
          pl.BlockSpec((_ROWS_BLK, HIDDEN), lambda i: (i, 0)),
          pl.BlockSpec((D_ATOM, HIDDEN), lambda i: (0, 0)),
          pl.BlockSpec((HIDDEN, HIDDEN), lambda i: (0, 0)),
          pl.BlockSpec((1, HIDDEN), lambda i: (0, 0)),
      ],
      out_specs=pl.BlockSpec((_ROWS_BLK, HIDDEN), lambda i: (i, 0)),
      out_shape=jax.ShapeDtypeStruct((N, HIDDEN), jnp.float32),
  )(x, h, W_o1, W_o2, b_o)


@jax.jit
def kernel(x, edge_index, edge_attr, W_in, b_in, W_edge, W_h, b_h, W_o, b_o):
  src = edge_index[0].astype(jnp.int32)
  dst = edge_index[1].astype(jnp.int32)
  pad = E_PAD - E
  src_p = jnp.concatenate([src, jnp.zeros((pad,), jnp.int32)])
  dst_p = jnp.concatenate([dst, jnp.full((pad,), N, jnp.int32)])
  src2d = src_p.reshape(E_PAD // CHUNK, CHUNK)
  dst2d = dst_p.reshape(E_PAD // CHUNK, CHUNK)
  ea_p = jnp.concatenate(
      [edge_attr, jnp.zeros((pad, D_BOND), jnp.float32)], axis=0)

  ea2 = _edge_segsum(ea_p, dst2d)
  h0, eagg = _tc_pre(x, ea2, W_in, b_in.reshape(1, HIDDEN), W_edge)

  h = h0
  for _ in range(3):
    acc = _gather_segsum(h, src2d, dst2d)
    h = _tc_step(acc, eagg, h0, W_h, b_h.reshape(1, HIDDEN))

  return _tc_out(x, h, W_o[:D_ATOM], W_o[D_ATOM:], b_o.reshape(1, HIDDEN))


# speedup vs baseline: 3.8098x; 1.5507x over previous
import functools

import jax
import jax.numpy as jnp
from jax import lax
from jax.experimental import pallas as pl
from jax.experimental.pallas import tpu as pltpu
from jax.experimental.pallas import tpu_sc as plsc

N = 10000
E = 320000
D_ATOM = 128
D_BOND = 16
HIDDEN = 128

NC = 2
NS = 16
NW = NC * NS

CHUNK = 128
CHUNKS_PER_TILE = 80
EDGES_PER_TILE = CHUNK * CHUNKS_PER_TILE
E_PAD = EDGES_PER_TILE * NW
N_ACC = N + 16
ZROWS_ACC = N_ACC // NS
OROWS = N // NS

_mesh = plsc.VectorSubcoreMesh(
    core_axis_name="c", subcore_axis_name="s", num_cores=NC, num_subcores=NS)


def _zero_vmem(ref, nrows, ncol):
  z = jnp.zeros((16,), jnp.float32)
  @pl.loop(0, nrows)
  def _(i):
    for j in range(ncol // 16):
      ref[i, pl.ds(j * 16, 16)] = z


NBUF = 4
HALF = HIDDEN // NC
CHUNKS_FS = E_PAD // (NS * CHUNK)


def _sc_gather_segsum(h_hbm, src_hbm, dst_hbm, out_hbm,
                      sidx, didx, rows, zbuf, acc, gsems, ssems):
  cid = lax.axis_index("c")
  sid = lax.axis_index("s")

  rbase = sid * CHUNKS_FS
  pltpu.sync_copy(src_hbm.at[pl.ds(rbase, CHUNKS_FS)], sidx)
  pltpu.sync_copy(dst_hbm.at[pl.ds(rbase, CHUNKS_FS)], didx)
  for b in range(NBUF):
    pltpu.async_copy(h_hbm.at[cid].at[sidx.at[b]], rows.at[b], gsems.at[b])

  _zero_vmem(zbuf, CHUNK, HALF)
  zbase = sid * ZROWS_ACC
  for k in range(ZROWS_ACC // CHUNK):
    pltpu.sync_copy(zbuf, acc.at[pl.ds(zbase + k * CHUNK, CHUNK)])
  rem = ZROWS_ACC % CHUNK
  if rem:
    pltpu.sync_copy(zbuf.at[pl.ds(0, rem)],
                    acc.at[pl.ds(zbase + (ZROWS_ACC // CHUNK) * CHUNK, rem)])

  plsc.subcore_barrier()

  @pl.loop(0, CHUNKS_FS - NBUF, step=NBUF)
  def _(j):
    for b in range(NBUF):
      pltpu.make_async_copy(h_hbm.at[cid].at[sidx.at[b]], rows.at[b],
                            gsems.at[b]).wait()
      pltpu.async_copy(rows.at[b], acc.at[didx.at[j + b]], ssems.at[b],
                       add=True)
    for b in range(NBUF):
      pltpu.make_async_copy(rows.at[b], acc.at[didx.at[j + b]],
                            ssems.at[b]).wait()
      pltpu.async_copy(h_hbm.at[cid].at[sidx.at[j + NBUF + b]], rows.at[b],
                       gsems.at[b])

  jlast = CHUNKS_FS - NBUF
  for b in range(NBUF):
    pltpu.make_async_copy(h_hbm.at[cid].at[sidx.at[b]], rows.at[b],
                          gsems.at[b]).wait()
    pltpu.async_copy(rows.at[b], acc.at[didx.at[jlast + b]], ssems.at[b],
                     add=True)
  for b in range(NBUF):
    pltpu.make_async_copy(rows.at[b], acc.at[didx.at[jlast + b]],
                          ssems.at[b]).wait()

  plsc.subcore_barrier()

  obase = sid * OROWS
  pltpu.sync_copy(acc.at[pl.ds(obase, OROWS)],
                  out_hbm.at[cid, pl.ds(obase, OROWS)])


_gather_segsum = pl.kernel(
    _sc_gather_segsum,
    out_type=jax.ShapeDtypeStruct((NC, N, HALF), jnp.float32),
    mesh=_mesh,
    compiler_params=pltpu.CompilerParams(use_tc_tiling_on_sc=False),
    scratch_types=[
        pltpu.VMEM((CHUNKS_FS, CHUNK), jnp.int32),
        pltpu.VMEM((CHUNKS_FS, CHUNK), jnp.int32),
        pltpu.VMEM((NBUF, CHUNK, HALF), jnp.float32),
        pltpu.VMEM((CHUNK, HALF), jnp.float32),
        pltpu.VMEM_SHARED((N_ACC, HALF), jnp.float32),
        pltpu.SemaphoreType.DMA((NBUF,)),
        pltpu.SemaphoreType.DMA((NBUF,)),
    ],
)


def _sc_edge_segsum(ea_hbm, dst_hbm, out_hbm, didx, rows, acc):
  cid = lax.axis_index("c")
  sid = lax.axis_index("s")
  wid = cid * NS + sid

  _zero_vmem(rows, CHUNK, D_BOND)
  zbase = sid * ZROWS_ACC
  for k in range(ZROWS_ACC // CHUNK):
    pltpu.sync_copy(rows, acc.at[pl.ds(zbase + k * CHUNK, CHUNK)])
  rem = ZROWS_ACC % CHUNK
  if rem:
    pltpu.sync_copy(rows.at[pl.ds(0, rem)],
                    acc.at[pl.ds(zbase + (ZROWS_ACC // CHUNK) * CHUNK, rem)])

  rbase = wid * CHUNKS_PER_TILE
  pltpu.sync_copy(dst_hbm.at[pl.ds(rbase, CHUNKS_PER_TILE)], didx)

  plsc.subcore_barrier()

  ebase = wid * EDGES_PER_TILE
  @pl.loop(0, CHUNKS_PER_TILE)
  def _(j):
    pltpu.sync_copy(ea_hbm.at[pl.ds(ebase + j * CHUNK, CHUNK)], rows)
    pltpu.sync_copy(rows, acc.at[didx.at[j]], add=True)

  plsc.subcore_barrier()

  obase = sid * OROWS
  pltpu.sync_copy(acc.at[pl.ds(obase, OROWS)],
                  out_hbm.at[cid, pl.ds(obase, OROWS)])


_edge_segsum = pl.kernel(
    _sc_edge_segsum,
    out_type=jax.ShapeDtypeStruct((NC, N, D_BOND), jnp.float32),
    mesh=_mesh,
    compiler_params=pltpu.CompilerParams(use_tc_tiling_on_sc=False),
    scratch_types=[
        pltpu.VMEM((CHUNKS_PER_TILE, CHUNK), jnp.int32),
        pltpu.VMEM((CHUNK, D_BOND), jnp.float32),
        pltpu.VMEM_SHARED((N_ACC, D_BOND), jnp.float32),
    ],
)


_ROWS_BLK = 1000
_GRID = N // _ROWS_BLK


def _split(v):
  return jnp.stack([v[:, :HALF], v[:, HALF:]], axis=0)


def _tc_pre_body(x_ref, ea_ref, win_ref, bin_ref, wedge_ref,
                 h0_ref, h0s_ref, eagg_ref):
  h0 = jnp.maximum(
      jnp.dot(x_ref[...], win_ref[...], preferred_element_type=jnp.float32)
      + bin_ref[...], 0.0)
  h0_ref[...] = h0
  h0s_ref[...] = _split(h0)
  ea = ea_ref[0] + ea_ref[1]
  eagg_ref[...] = jnp.dot(ea, wedge_ref[...],
                          preferred_element_type=jnp.float32)


def _tc_pre(x, ea2, W_in, b_in, W_edge):
  return pl.pallas_call(
      _tc_pre_body,
      grid=(_GRID,),
      in_specs=[
          pl.BlockSpec((_ROWS_BLK, D_ATOM), lambda i: (i, 0)),
          pl.BlockSpec((NC, _ROWS_BLK, D_BOND), lambda i: (0, i, 0)),
          pl.BlockSpec((D_ATOM, HIDDEN), lambda i: (0, 0)),
          pl.BlockSpec((1, HIDDEN), lambda i: (0, 0)),
          pl.BlockSpec((D_BOND, HIDDEN), lambda i: (0, 0)),
      ],
      out_specs=[
          pl.BlockSpec((_ROWS_BLK, HIDDEN), lambda i: (i, 0)),
          pl.BlockSpec((NC, _ROWS_BLK, HALF), lambda i: (0, i, 0)),
          pl.BlockSpec((_ROWS_BLK, HIDDEN), lambda i: (i, 0)),
      ],
      out_shape=[
          jax.ShapeDtypeStruct((N, HIDDEN), jnp.float32),
          jax.ShapeDtypeStruct((NC, N, HALF), jnp.float32),
          jax.ShapeDtypeStruct((N, HIDDEN), jnp.float32),
      ],
  )(x, ea2, W_in, b_in, W_edge)


def _tc_step_body(acc_ref, eagg_ref, h0_ref, wh_ref, bh_ref, h_ref):
  agg = jnp.concatenate([acc_ref[0], acc_ref[1]], axis=1) + eagg_ref[...]
  h = jnp.maximum(
      jnp.dot(agg, wh_ref[...], preferred_element_type=jnp.float32)
      + bh_ref[...] + h0_ref[...], 0.0)
  h_ref[...] = _split(h)


def _tc_step(acc, eagg, h0, W_h, b_h):
  return pl.pallas_call(
      _tc_step_body,
      grid=(_GRID,),
      in_specs=[
          pl.BlockSpec((NC, _ROWS_BLK, HALF), lambda i: (0, i, 0)),
          pl.BlockSpec((_ROWS_BLK, HIDDEN), lambda i: (i, 0)),
          pl.BlockSpec((_ROWS_BLK, HIDDEN), lambda i: (i, 0)),
          pl.BlockSpec((HIDDEN, HIDDEN), lambda i: (0, 0)),
          pl.BlockSpec((1, HIDDEN), lambda i: (0, 0)),
      ],
      out_specs=pl.BlockSpec((NC, _ROWS_BLK, HALF), lambda i: (0, i, 0)),
      out_shape=jax.ShapeDtypeStruct((NC, N, HALF), jnp.float32),
  )(acc, eagg, h0, W_h, b_h)


def _tc_out_body(x_ref, h_ref, wo1_ref, wo2_ref, bo_ref, out_ref):
  h = jnp.concatenate([h_ref[0], h_ref[1]], axis=1)
  out_ref[...] = jnp.maximum(
      jnp.dot(x_ref[...], wo1_ref[...], preferred_element_type=jnp.float32)
      + jnp.dot(h, wo2_ref[...], preferred_element_type=jnp.float32)
      + bo_ref[...], 0.0)


def _tc_out(x, h2, W_o1, W_o2, b_o):
  return pl.pallas_call(
      _tc_out_body,
      grid=(_GRID,),
      in_specs=[
          pl.BlockSpec((_ROWS_BLK, D_ATOM), lambda i: (i, 0)),
          pl.BlockSpec((NC, _ROWS_BLK, HALF), lambda i: (0, i, 0)),
          pl.BlockSpec((D_ATOM, HIDDEN), lambda i: (0, 0)),
          pl.BlockSpec((HIDDEN, HIDDEN), lambda i: (0, 0)),
          pl.BlockSpec((1, HIDDEN), lambda i: (0, 0)),
      ],
      out_specs=pl.BlockSpec((_ROWS_BLK, HIDDEN), lambda i: (i, 0)),
      out_shape=jax.ShapeDtypeStruct((N, HIDDEN), jnp.float32),
  )(x, h2, W_o1, W_o2, b_o)


@jax.jit
def kernel(x, edge_index, edge_attr, W_in, b_in, W_edge, W_h, b_h, W_o, b_o):
  src = edge_index[0].astype(jnp.int32)
  dst = edge_index[1].astype(jnp.int32)
  pad = E_PAD - E
  src_p = jnp.concatenate([src, jnp.zeros((pad,), jnp.int32)])
  dst_p = jnp.concatenate([dst, jnp.full((pad,), N, jnp.int32)])
  src2d = src_p.reshape(E_PAD // CHUNK, CHUNK)
  dst2d = dst_p.reshape(E_PAD // CHUNK, CHUNK)
  ea_p = jnp.concatenate(
      [edge_attr, jnp.zeros((pad, D_BOND), jnp.float32)], axis=0)

  ea2 = _edge_segsum(ea_p, dst2d)
  h0, h2, eagg = _tc_pre(x, ea2, W_in, b_in.reshape(1, HIDDEN), W_edge)

  for _ in range(3):
    acc = _gather_segsum(h2, src2d, dst2d)
    h2 = _tc_step(acc, eagg, h0, W_h, b_h.reshape(1, HIDDEN))

  return _tc_out(x, h2, W_o[:D_ATOM], W_o[D_ATOM:], b_o.reshape(1, HIDDEN))
